# Initial kernel scaffold; baseline (speedup 1.0000x reference)
import jax
import jax.numpy as jnp
from jax.experimental import pallas as pl


def kernel(static_cont, temporal, region_ids, state_ids, nlcd_ids, W_static, b_static, g_static, be_static, W_temp, b_temp, g_temp, be_temp, region_table, state_table, nlcd_table, W_cat, b_cat, temporal_pos, W_combine, b_combine, g_out, be_out):
    raise NotImplementedError("write your pallas kernel here")


# speedup vs baseline: 2.0153x; 2.0153x over previous
import functools

import jax
import jax.numpy as jnp
from jax import lax
from jax.experimental import pallas as pl
from jax.experimental.pallas import tpu as pltpu
from jax.experimental.pallas import tpu_sc as plsc

_BT = 512


def _sc_region_gather(table, idx):
    B = idx.shape[0]
    D = table.shape[1]
    info = plsc.get_sparse_core_info()
    nw = info.num_cores * info.num_subcores
    b_per_w = B // nw
    mesh = plsc.VectorSubcoreMesh(core_axis_name="c", subcore_axis_name="s")

    @functools.partial(
        pl.kernel,
        mesh=mesh,
        out_type=jax.ShapeDtypeStruct((B, D), jnp.float32),
        scratch_types=[
            pltpu.VMEM((b_per_w,), jnp.int32),
            pltpu.VMEM((b_per_w, D), jnp.float32),
            pltpu.SemaphoreType.DMA,
        ],
        compiler_params=pltpu.CompilerParams(use_tc_tiling_on_sc=False),
    )
    def gather_kernel(table_hbm, idx_hbm, out_hbm, idx_v, rows_v, sem):
        wid = lax.axis_index("s") * info.num_cores + lax.axis_index("c")
        base = wid * b_per_w
        pltpu.sync_copy(idx_hbm.at[pl.ds(base, b_per_w)], idx_v)
        pltpu.async_copy(table_hbm.at[idx_v], rows_v, sem).wait()
        pltpu.sync_copy(rows_v, out_hbm.at[pl.ds(base, b_per_w)])

    return gather_kernel(table, idx)


def _ln(x, g, b, eps=1e-5):
    m = jnp.mean(x, axis=-1, keepdims=True)
    v = jnp.mean((x - m) * (x - m), axis=-1, keepdims=True)
    return (x - m) / jnp.sqrt(v + eps) * g + b


def _gelu(x):
    return 0.5 * x * (1.0 + lax.erf(x * 0.7071067811865476))


def _tc_body(sc_ref, tmp_ref, reg_ref, sid_ref, nid_ref,
             Ws_ref, bs_ref, gs_ref, bes_ref,
             Wt_ref, bt_ref, gt_ref, bet_ref,
             stab_ref, ntab_ref, Wc_ref, bc_ref, pos_ref,
             Wcb_ref, bcb_ref, go_ref, beo_ref, out_ref):
    f32 = jnp.float32
    bt = sc_ref.shape[0]

    s_emb = _gelu(_ln(jnp.dot(sc_ref[...], Ws_ref[...],
                              preferred_element_type=f32) + bs_ref[...],
                      gs_ref[...], bes_ref[...]))

    tmp = tmp_ref[...]
    Wt = Wt_ref[...]
    btv = bt_ref[...]
    gtv = gt_ref[...]
    betv = bet_ref[...]
    acc = jnp.zeros((bt, 128), f32)
    for t in range(14):
        xt = tmp[:, 20 * t:20 * t + 20]
        acc = acc + _gelu(_ln(jnp.dot(xt, Wt, preferred_element_type=f32)
                              + btv, gtv, betv))
    pos_mean = jnp.mean(pos_ref[...], axis=0, keepdims=True)
    pooled = acc * (1.0 / 14.0) + pos_mean

    Wc = Wc_ref[...]
    sproj = jnp.dot(stab_ref[...], Wc[64:96, :], preferred_element_type=f32)
    nproj = jnp.dot(ntab_ref[...], Wc[96:128, :], preferred_element_type=f32)
    oh_s = (sid_ref[...] == lax.broadcasted_iota(jnp.int32, (bt, 8), 1)
            ).astype(f32)
    oh_n = (nid_ref[...] == lax.broadcasted_iota(jnp.int32, (bt, 24), 1)
            ).astype(f32)
    cat = (jnp.dot(reg_ref[...], Wc[0:64, :], preferred_element_type=f32)
           + jnp.dot(oh_s[:, :5], sproj, preferred_element_type=f32)
           + jnp.dot(oh_n[:, :20], nproj, preferred_element_type=f32)
           + bc_ref[...])

    Wcb = Wcb_ref[...]
    out = (jnp.dot(s_emb, Wcb[0:128, :], preferred_element_type=f32)
           + jnp.dot(pooled, Wcb[128:256, :], preferred_element_type=f32)
           + jnp.dot(cat, Wcb[256:384, :], preferred_element_type=f32)
           + bcb_ref[...])
    out_ref[...] = _ln(out, go_ref[...], beo_ref[...])


def _tc_fused(static_cont, temporal2d, region_emb, sid2, nid2,
              W_static, b_static, g_static, be_static,
              W_temp, b_temp, g_temp, be_temp,
              state_table, nlcd_table, W_cat, b_cat, pos2d,
              W_combine, b_combine, g_out, be_out):
    B = static_cont.shape[0]
    grid = (B // _BT,)

    def row_spec(cols):
        return pl.BlockSpec((_BT, cols), lambda i: (i, 0))

    def full_spec(shape):
        return pl.BlockSpec(shape, lambda i: tuple(0 for _ in shape))

    in_specs = [
        row_spec(50),
        row_spec(280),
        row_spec(64),
        row_spec(1),
        row_spec(1),
        full_spec((50, 128)),
        full_spec((1, 128)),
        full_spec((1, 128)),
        full_spec((1, 128)),
        full_spec((20, 128)),
        full_spec((1, 128)),
        full_spec((1, 128)),
        full_spec((1, 128)),
        full_spec((5, 32)),
        full_spec((20, 32)),
        full_spec((128, 128)),
        full_spec((1, 128)),
        full_spec((14, 128)),
        full_spec((384, 128)),
        full_spec((1, 128)),
        full_spec((1, 128)),
        full_spec((1, 128)),
    ]
    return pl.pallas_call(
        _tc_body,
        grid=grid,
        in_specs=in_specs,
        out_specs=pl.BlockSpec((_BT, 128), lambda i: (i, 0)),
        out_shape=jax.ShapeDtypeStruct((B, 128), jnp.float32),
    )(static_cont, temporal2d, region_emb, sid2, nid2,
      W_static, b_static, g_static, be_static,
      W_temp, b_temp, g_temp, be_temp,
      state_table, nlcd_table, W_cat, b_cat, pos2d,
      W_combine, b_combine, g_out, be_out)


def kernel(static_cont, temporal, region_ids, state_ids, nlcd_ids,
           W_static, b_static, g_static, be_static,
           W_temp, b_temp, g_temp, be_temp,
           region_table, state_table, nlcd_table,
           W_cat, b_cat, temporal_pos,
           W_combine, b_combine, g_out, be_out):
    B = static_cont.shape[0]
    region_emb = _sc_region_gather(region_table,
                                   region_ids.astype(jnp.int32))
    row = lambda v: v.reshape(1, -1)
    return _tc_fused(
        static_cont,
        temporal.reshape(B, 14 * 20),
        region_emb,
        state_ids.astype(jnp.int32).reshape(B, 1),
        nlcd_ids.astype(jnp.int32).reshape(B, 1),
        W_static, row(b_static), row(g_static), row(be_static),
        W_temp, row(b_temp), row(g_temp), row(be_temp),
        state_table, nlcd_table, W_cat, row(b_cat),
        temporal_pos.reshape(14, 128),
        W_combine, row(b_combine), row(g_out), row(be_out))
